# baseline (speedup 1.0000x reference)
import functools

import jax
import jax.numpy as jnp
from jax import lax
from jax.experimental import pallas as pl
from jax.experimental.pallas import tpu as pltpu
from jax.experimental.pallas import tpu_sc as plsc

_NC = 2
_NS = 16
_NW = _NC * _NS
_LANES = 16
_C = 64


def _project_table(values, wp, m_block):
    n_coarse, d = values.shape
    n_out = wp.shape[1]

    def body(v_ref, w_ref, o_ref):
        o_ref[...] = lax.dot_general(
            v_ref[...], w_ref[...], (((1,), (0,)), ((), ())),
            preferred_element_type=jnp.float32,
            precision=lax.Precision.HIGHEST)

    return pl.pallas_call(
        body,
        grid=(n_coarse // m_block,),
        in_specs=[
            pl.BlockSpec((m_block, d), lambda i: (i, 0)),
            pl.BlockSpec((d, n_out), lambda i: (0, 0)),
        ],
        out_specs=pl.BlockSpec((m_block, n_out), lambda i: (i, 0)),
        out_shape=jax.ShapeDtypeStruct((n_coarse, n_out), jnp.float32),
    )(values, wp)


def _gather_sum(table, idx_chunks, fe, nf, n_chunks, n_pad):
    assert n_chunks % 2 == 0
    mesh = plsc.VectorSubcoreMesh(core_axis_name="c", subcore_axis_name="s")

    @functools.partial(
        pl.kernel,
        out_type=jax.ShapeDtypeStruct((n_pad, nf), jnp.float32),
        mesh=mesh,
        scratch_types=[
            pltpu.VMEM((fe, _C), jnp.int32),
            pltpu.VMEM((fe, _C), jnp.int32),
            pltpu.VMEM((fe, _C, nf), jnp.float32),
            pltpu.VMEM((fe, _C, nf), jnp.float32),
            pltpu.VMEM((_C, nf), jnp.float32),
            pltpu.SemaphoreType.DMA,
            pltpu.SemaphoreType.DMA,
            pltpu.SemaphoreType.DMA,
            pltpu.SemaphoreType.DMA,
        ],
        compiler_params=pltpu.CompilerParams(use_tc_tiling_on_sc=False),
    )
    def body(table_hbm, idx_hbm, out_hbm,
             idx0, idx1, rows0, rows1, acc_v, semi0, semi1, semr0, semr1):
        wid = lax.axis_index("s") * _NC + lax.axis_index("c")
        base = wid * n_chunks
        idx_b = (idx0, idx1)
        rows_b = (rows0, rows1)
        semi = (semi0, semi1)
        semr = (semr0, semr1)

        def fire_gathers(p):
            for k in range(fe):
                pltpu.async_copy(table_hbm.at[idx_b[p].at[k]],
                                 rows_b[p].at[k], semr[p])

        def wait_gathers(p):
            for k in range(fe):
                pltpu.make_async_copy(table_hbm.at[idx_b[p].at[k]],
                                      rows_b[p].at[k], semr[p]).wait()

        def wait_idx(p):
            pltpu.make_async_copy(idx_hbm.at[base], idx_b[p], semi[p]).wait()

        def accumulate_and_store(p, c):
            @pl.loop(0, _C)
            def _row(i):
                for j in range(nf // _LANES):
                    s = pl.ds(j * _LANES, _LANES)
                    v = rows_b[p][0, i, s]
                    for k in range(1, fe):
                        v = v + rows_b[p][k, i, s]
                    acc_v[i, s] = v

            pltpu.sync_copy(acc_v, out_hbm.at[pl.ds((base + c) * _C, _C)])

        pltpu.sync_copy(idx_hbm.at[base], idx0)
        fire_gathers(0)
        pltpu.async_copy(idx_hbm.at[base + 1], idx1, semi1)

        @pl.loop(0, n_chunks, step=2)
        def _pair(c):
            wait_idx(1)
            fire_gathers(1)
            wait_gathers(0)

            @pl.when(c + 2 < n_chunks)
            def _():
                pltpu.async_copy(idx_hbm.at[base + c + 2], idx0, semi0)

            accumulate_and_store(0, c)

            @pl.when(c + 2 < n_chunks)
            def _():
                wait_idx(0)
                fire_gathers(0)

            wait_gathers(1)

            @pl.when(c + 3 < n_chunks)
            def _():
                pltpu.async_copy(idx_hbm.at[base + c + 3], idx1, semi1)

            accumulate_and_store(1, c + 1)

    return body(table, idx_chunks)


def kernel(lattice_coarse_values, neighbor_indices, weight):
    n_coarse, d = lattice_coarse_values.shape
    n_fine, fe = neighbor_indices.shape
    nf = weight.shape[1]

    wp = weight.reshape(fe, d, nf).transpose(1, 0, 2).reshape(d, fe * nf)
    p2 = _project_table(lattice_coarse_values, wp, m_block=1000)
    table = p2.reshape(n_coarse * fe, nf)

    per_round = _NW * _C
    n_chunks = -(-n_fine // per_round)
    n_chunks += n_chunks % 2
    n_pad = n_chunks * per_round
    idx32 = neighbor_indices.astype(jnp.int32)
    flat_idx = idx32 * fe + jnp.arange(fe, dtype=jnp.int32)[None, :]
    idx_t = jnp.pad(flat_idx.T, ((0, 0), (0, n_pad - n_fine)))
    idx_chunks = idx_t.reshape(fe, _NW * n_chunks, _C).transpose(1, 0, 2)

    out = _gather_sum(table, idx_chunks, fe, nf, n_chunks, n_pad)
    return out[:n_fine]


# speedup vs baseline: 2.1073x; 2.1073x over previous
import functools

import jax
import jax.numpy as jnp
from jax import lax
from jax.experimental import pallas as pl
from jax.experimental.pallas import tpu as pltpu
from jax.experimental.pallas import tpu_sc as plsc

_NC = 2
_NS = 16
_NW = _NC * _NS
_LANES = 16
_C = 64


def _project_table(values, w9, m_block):
    n_coarse, d = values.shape
    fe, _, nf = w9.shape

    def body(v_ref, w_ref, o_ref):
        o_ref[0] = lax.dot_general(
            v_ref[...], w_ref[0], (((1,), (0,)), ((), ())),
            preferred_element_type=jnp.float32,
            precision=lax.Precision.HIGHEST).astype(jnp.bfloat16)

    return pl.pallas_call(
        body,
        grid=(n_coarse // m_block, fe),
        in_specs=[
            pl.BlockSpec((m_block, d), lambda m, k: (m, 0)),
            pl.BlockSpec((1, d, nf), lambda m, k: (k, 0, 0)),
        ],
        out_specs=pl.BlockSpec((1, m_block, nf), lambda m, k: (k, m, 0)),
        out_shape=jax.ShapeDtypeStruct((fe, n_coarse, nf), jnp.bfloat16),
    )(values, w9)


def _gather_sum(table, idx_w, fe, nf, n_coarse, bpw, n_pad):
    n_ch = bpw // _C
    assert n_ch % 2 == 0
    rows_a = (n_coarse // _NS) & ~7
    rows_b = n_coarse - _NS * rows_a
    mesh = plsc.VectorSubcoreMesh(core_axis_name="c", subcore_axis_name="s")

    @functools.partial(
        pl.kernel,
        out_type=jax.ShapeDtypeStruct((n_pad, nf), jnp.bfloat16),
        mesh=mesh,
        scratch_types=[
            pltpu.VMEM_SHARED((n_coarse, nf), jnp.bfloat16),
            pltpu.VMEM_SHARED((n_coarse, nf), jnp.bfloat16),
            pltpu.VMEM((bpw, nf), jnp.bfloat16),
            pltpu.VMEM((_C, nf), jnp.bfloat16),
            pltpu.VMEM((_C, nf), jnp.bfloat16),
            pltpu.VMEM((bpw,), jnp.int32),
            pltpu.VMEM((bpw,), jnp.int32),
            pltpu.SemaphoreType.DMA,
            pltpu.SemaphoreType.DMA,
            pltpu.SemaphoreType.DMA,
            pltpu.SemaphoreType.DMA,
            pltpu.SemaphoreType.DMA,
            pltpu.SemaphoreType.DMA,
        ],
        compiler_params=pltpu.CompilerParams(use_tc_tiling_on_sc=False),
    )
    def body(table_hbm, idx_hbm, out_hbm,
             sh0, sh1, acc_v, r0, r1, ix0, ix1,
             ss0, ss1, sg0, sg1, si0, si1):
        cid = lax.axis_index("c")
        sid = lax.axis_index("s")
        wid = sid * _NC + cid
        sh = (sh0, sh1)
        rows = (r0, r1)
        ix = (ix0, ix1)
        ss = (ss0, ss1)
        sg = (sg0, sg1)
        si = (si0, si1)

        def stage_copies(k, q):
            a = sid * rows_a
            yield (table_hbm.at[k, pl.ds(a, rows_a)],
                   sh[q].at[pl.ds(a, rows_a)], ss[q])
            b = _NS * rows_a
            yield (table_hbm.at[k, pl.ds(b, rows_b)],
                   sh[q].at[pl.ds(b, rows_b)], ss[q])

        def stage_start(k, q):
            cps = list(stage_copies(k, q))
            pltpu.async_copy(*cps[0])

            @pl.when(sid == 0)
            def _():
                pltpu.async_copy(*cps[1])

        def stage_wait(k, q):
            cps = list(stage_copies(k, q))
            pltpu.make_async_copy(*cps[0]).wait()

            @pl.when(sid == 0)
            def _():
                pltpu.make_async_copy(*cps[1]).wait()

        def idx_start(k, p):
            pltpu.async_copy(idx_hbm.at[wid, k], ix[p], si[p])

        def idx_wait(k, p):
            pltpu.make_async_copy(idx_hbm.at[wid, k], ix[p], si[p]).wait()

        def gather(k, q, c, p):
            src = sh[q].at[ix[k % 2].at[pl.ds(c * _C, _C)]]
            if k == 0:
                pltpu.async_copy(src, acc_v.at[pl.ds(c * _C, _C)], sg[p])
            else:
                pltpu.async_copy(src, rows[p], sg[p])

        def gather_wait(k, q, p):
            src = sh[q].at[ix[k % 2].at[pl.ds(0, _C)]]
            if k == 0:
                pltpu.make_async_copy(src, acc_v.at[pl.ds(0, _C)], sg[p]).wait()
            else:
                pltpu.make_async_copy(src, rows[p], sg[p]).wait()

        def accumulate(k, c, p):
            if k == 0:
                return

            @pl.loop(0, _C, step=2)
            def _row(r):
                for rr in range(2):
                    for j in range(nf // (2 * _LANES)):
                        s = pl.ds(j * 2 * _LANES, 2 * _LANES)
                        row = c * _C + r + rr
                        acc_v[row, s] = acc_v[row, s] + rows[p][r + rr, s]

        stage_start(0, 0)
        stage_start(1, 1)
        idx_start(0, 0)
        idx_start(1, 1)
        stage_wait(0, 0)
        idx_wait(0, 0)
        plsc.subcore_barrier()

        for k in range(fe):
            q = k % 2
            gather(k, q, 0, 0)

            @pl.loop(0, n_ch, step=2)
            def _pair(c, k=k, q=q):
                gather_wait(k, q, 0)
                gather(k, q, c + 1, 1)
                accumulate(k, c, 0)
                gather_wait(k, q, 1)

                @pl.when(c + 2 < n_ch)
                def _():
                    gather(k, q, c + 2, 0)

                accumulate(k, c + 1, 1)

            if k + 2 <= fe - 1:
                idx_start(k + 2, q)
            if k + 1 <= fe - 1:
                stage_wait(k + 1, 1 - q)
                idx_wait(k + 1, 1 - q)
                plsc.subcore_barrier()
                if k + 2 <= fe - 1:
                    stage_start(k + 2, q)

        pltpu.sync_copy(acc_v, out_hbm.at[pl.ds(wid * bpw, bpw)])

    return body(table, idx_w)


def kernel(lattice_coarse_values, neighbor_indices, weight):
    n_coarse, d = lattice_coarse_values.shape
    n_fine, fe = neighbor_indices.shape
    nf = weight.shape[1]

    w9 = weight.reshape(fe, d, nf)
    table = _project_table(lattice_coarse_values, w9, m_block=2000)

    per_round = _NW * _C
    n_chunks = -(-n_fine // per_round)
    n_chunks += n_chunks % 2
    n_pad = n_chunks * per_round
    bpw = n_chunks * _C
    idx32 = neighbor_indices.astype(jnp.int32)
    idx_t = jnp.pad(idx32.T, ((0, 0), (0, n_pad - n_fine)))
    idx_w = idx_t.reshape(fe, _NW, bpw).transpose(1, 0, 2)

    out = _gather_sum(table, idx_w, fe, nf, n_coarse, bpw, n_pad)
    return out[:n_fine].astype(jnp.float32)
